# Initial kernel scaffold; baseline (speedup 1.0000x reference)
import jax
import jax.numpy as jnp
from jax.experimental import pallas as pl


def kernel(t, x, fourier_coeffs, qW1, qb1, qW2, qb2, cW1, cb1, cW2, cb2, tW1, tb1, tW2, tb2, hW1, hb1, hW2, hb2, edge_index, enso_edge_index):
    raise NotImplementedError("write your pallas kernel here")


# speedup vs baseline: 177.7543x; 177.7543x over previous
import numpy as np
import jax
import jax.numpy as jnp
from jax.experimental import pallas as pl
from jax.experimental.pallas import tpu as pltpu


def _odefunc_kernel(scal_ref, x_ref, fcT_ref,
                    qW1_ref, qb1_ref, qW2_ref, qb2_ref,
                    cW1_ref, cb1_ref, cW2_ref, cb2_ref,
                    tW1_ref, tb1_ref, tW2_ref,
                    hW1_ref, hb1_ref, hW2_ref,
                    out_ref):
    c1 = scal_ref[0]
    s1 = scal_ref[1]
    c2 = scal_ref[2]
    s2 = scal_ref[3]
    tb2 = scal_ref[4]
    hb2 = scal_ref[5]

    x = x_ref[:, :]

    LT = (fcT_ref[0] + c1 * fcT_ref[1] + s1 * fcT_ref[2]
          + c2 * fcT_ref[3] + s2 * fcT_ref[4])
    linear = jnp.dot(x, LT, preferred_element_type=jnp.float32)

    hq = jnp.maximum(
        jnp.dot(x, qW1_ref[:, :], preferred_element_type=jnp.float32)
        + qb1_ref[0:1, :], 0.0)
    hc = jnp.maximum(
        jnp.dot(x, cW1_ref[:, :], preferred_element_type=jnp.float32)
        + cb1_ref[0:1, :], 0.0)
    sq = jnp.mean(
        jnp.dot(hq, qW2_ref[:, :], preferred_element_type=jnp.float32)
        + qb2_ref[0:1, :], axis=1, keepdims=True)
    sc = jnp.mean(
        jnp.dot(hc, cW2_ref[:, :], preferred_element_type=jnp.float32)
        + cb2_ref[0:1, :], axis=1, keepdims=True)
    s = sq + sc

    T = x[:, 0:1]
    Hh = x[:, 1:2]
    T2 = T * T
    TH = T * Hh
    gT = (T * tW1_ref[0:1, :] + Hh * tW1_ref[1:2, :] + T2 * tW1_ref[2:3, :]
          + TH * tW1_ref[3:4, :] + (T2 * T) * tW1_ref[4:5, :] + tb1_ref[0:1, :])
    eT = jnp.sum(jnp.maximum(gT, 0.0) * tW2_ref[0:1, :],
                 axis=1, keepdims=True) + tb2
    gH = (T * hW1_ref[0:1, :] + Hh * hW1_ref[1:2, :] + T2 * hW1_ref[2:3, :]
          + TH * hW1_ref[3:4, :] + (TH * Hh) * hW1_ref[4:5, :] + hb1_ref[0:1, :])
    eH = jnp.sum(jnp.maximum(gH, 0.0) * hW2_ref[0:1, :],
                 axis=1, keepdims=True) + hb2

    col = jax.lax.broadcasted_iota(jnp.int32, x.shape, 1)
    out_ref[:, :] = (linear + s
                     + jnp.where(col == 0, eT, 0.0)
                     + jnp.where(col == 1, eH, 0.0))


def kernel(t, x, fourier_coeffs, qW1, qb1, qW2, qb2, cW1, cb1, cW2, cb2,
           tW1, tb1, tW2, tb2, hW1, hb1, hW2, hb2, edge_index, enso_edge_index):
    omega = np.float32(2.0 * np.pi / 12.0)
    ts = t[0]
    scal = jnp.stack([jnp.cos(omega * ts), jnp.sin(omega * ts),
                      jnp.cos(2.0 * omega * ts), jnp.sin(2.0 * omega * ts),
                      tb2[0], hb2[0]]).astype(jnp.float32)
    fcT = jnp.transpose(fourier_coeffs, (2, 1, 0))

    in_specs = [pl.BlockSpec(memory_space=pltpu.SMEM)] + [
        pl.BlockSpec(memory_space=pltpu.VMEM) for _ in range(16)]

    return pl.pallas_call(
        _odefunc_kernel,
        out_shape=jax.ShapeDtypeStruct(x.shape, jnp.float32),
        in_specs=in_specs,
        out_specs=pl.BlockSpec(memory_space=pltpu.VMEM),
    )(scal, x, fcT,
      qW1, qb1[None, :], qW2, qb2[None, :],
      cW1, cb1[None, :], cW2, cb2[None, :],
      tW1, tb1[None, :], tW2.T,
      hW1, hb1[None, :], hW2.T)
